# baseline (speedup 1.0000x reference)
import math

import jax
import jax.numpy as jnp
from jax import lax
from jax.experimental import pallas as pl
from jax.experimental.pallas import tpu as pltpu

N = 50000
AT = 50
B = 1000
K = 8
IN_F = 16
TEMB = 16
HID = 128
OUT_F = 64
NG = 20
DEPTH = 2
NT = 1000

G = 8
V = G * AT
E = V * K
NBLK = B // G

_f32 = jnp.float32
_bf16 = jnp.bfloat16


def _mm(a, b):
    return lax.dot_general(a, b, (((1,), (0,)), ((), ())),
                           preferred_element_type=_f32)


def _hilo(v):
    vh = v.astype(_bf16)
    return vh, v - vh.astype(_f32)


def _mm_sel(oh_b, vals):
    f = vals.shape[1]
    v1, r1 = _hilo(vals)
    v2, r2 = _hilo(r1)
    v3 = r2.astype(_bf16)
    dn = (((1,), (0,)), ((), ()))
    if 3 * f <= 256:
        s = lax.dot_general(oh_b, jnp.concatenate([v1, v2, v3], axis=1),
                            dn, preferred_element_type=_f32)
        return s[:, :f] + s[:, f:2 * f] + s[:, 2 * f:]
    s = lax.dot_general(oh_b, jnp.concatenate([v1, v2], axis=1),
                        dn, preferred_element_type=_f32)
    return (s[:, :f] + s[:, f:]
            + lax.dot_general(oh_b, v3, dn, preferred_element_type=_f32))


def _silu(v):
    return v * jax.nn.sigmoid(v)


def _ln(v, g, b):
    m = jnp.mean(v, axis=-1, keepdims=True)
    d = v - m
    var = jnp.mean(d * d, axis=-1, keepdims=True)
    inv = 1.0 / jnp.sqrt(var + 1e-5)
    return d * inv * g + b


def _iota(shape, dim):
    return lax.broadcasted_iota(jnp.int32, shape, dim).astype(_f32)


def _body(*refs):
    xcb = refs[0][...]
    ysw = refs[1][0]
    hb = refs[2][...]
    tfb = refs[3][...]
    tt = refs[4][...]
    na = refs[5][0, 0]
    w = [r[...] for r in refs[8:-1]]
    out_ref = refs[-1]

    rowid = _iota((V, 1), 0)
    ligr = jnp.floor((rowid + 0.5) * (1.0 / AT))
    aloc = rowid - AT * ligr
    iota_ca = _iota((V, AT), 1)
    iota_cv = _iota((V, V), 1)

    iott = _iota((V, NT), 1)
    temb = _mm_sel((iott == tfb).astype(_bf16), tt)

    ohg = (_iota((V, G), 1) == ligr).astype(_bf16)
    d2 = None
    for c in range(3):
        yg = _mm_sel(ohg, ysw[c * G:(c + 1) * G])
        dd = xcb[:, c:c + 1] - yg
        d2 = dd * dd if d2 is None else d2 + dd * dd
    d2m = d2 + jnp.where(iota_ca == aloc, 1e9, 0.0)

    oh_list, rad_list = [], []
    for _ in range(K):
        minv = jnp.min(d2m, axis=1, keepdims=True)
        idx = jnp.min(jnp.where(d2m == minv, iota_ca, 1e9),
                      axis=1, keepdims=True)
        sel = iota_ca == idx
        idxg = idx + AT * ligr
        oh_list.append((iota_cv == idxg).astype(_bf16))
        rad_list.append(minv)
        d2m = jnp.where(sel, 1e30, d2m)
    ohs = jnp.concatenate(oh_list, axis=0)
    radial = jnp.concatenate(rad_list, axis=0)
    keep = [(r < 7.0).astype(_f32) for r in rad_list]

    off = refs[6][...]
    coeff = refs[7][...]
    dc = jnp.clip(radial, 0.0, 4.0) - off
    smear = jnp.exp(coeff * dc * dc)

    it = iter(w)
    emb_in_w, emb_b = next(it), next(it)
    z = _mm(jnp.concatenate([hb, temb], axis=1), emb_in_w) + emb_b

    for _ in range(DEPTH):
        (e1_w, e1_b, ln1_g, ln1_b, e2_w, e2_b, att_w, att_b,
         n1_w, n1_b, ln2_g, ln2_b, n2_w, n2_b) = (
            next(it) for _ in range(14))
        common = (_mm(z, e1_w[0:HID]) + _mm(temb, e1_w[2 * HID + NG:])
                  + e1_b)
        gath = _mm_sel(ohs, _mm(z, e1_w[HID:2 * HID]))
        m = (jnp.concatenate([common] * K, axis=0) + gath
             + _mm(smear, e1_w[2 * HID:2 * HID + NG]))
        m = _silu(_ln(m, ln1_g, ln1_b))
        mij = _silu(_mm(m, e2_w) + e2_b)
        alog = jnp.concatenate(
            [_mm(mij[k * V:(k + 1) * V], att_w) for k in range(K)],
            axis=1) + att_b
        att = jax.nn.sigmoid(alog)
        agg = None
        for k in range(K):
            gk = mij[k * V:(k + 1) * V] * (att[:, k:k + 1] * keep[k])
            agg = gk if agg is None else agg + gk
        agg = agg / 5.0
        o = _silu(_ln(_mm(jnp.concatenate([z, agg], axis=1), n1_w)
                      + n1_b, ln2_g, ln2_b))
        z = z + _mm(o, n2_w) + n2_b

    emb_out_w, emb_out_b, out_w, out_b = (next(it) for _ in range(4))
    zo = _mm(z, emb_out_w) + emb_out_b
    pmat = (_iota((G, V), 0)
            == jnp.floor((_iota((G, V), 1) + 0.5) * (1.0 / AT))
            ).astype(_bf16)
    pooled = _mm_sel(pmat, zo) / na
    out_ref[0] = _mm(pooled, out_w) + out_b


def kernel(x, h, t, num_atoms_per_ligand, batch_ligand, params, time_table):
    xr = x.reshape(NBLK, G, AT, 3)
    ysw = xr.transpose(0, 3, 1, 2).reshape(NBLK, 3 * G, AT)
    tf = t.astype(_f32).reshape(N, 1)
    na = jnp.asarray(num_atoms_per_ligand, _f32).reshape(1, 1)

    off1 = jnp.exp(jnp.linspace(jnp.log(1.0), jnp.log(5.0), NG)) - 1.0
    df = jnp.diff(off1)
    df = jnp.concatenate([df[:1], df])
    coeff1 = -0.5 / df ** 2
    off = off1.reshape(1, NG).astype(_f32)
    coeff = coeff1.reshape(1, NG).astype(_f32)

    p = params
    weights = [p['emb_in_W'], p['emb_in_b'].reshape(1, HID)]
    for lp in p['layers']:
        weights += [
            lp['e1_W'], lp['e1_b'].reshape(1, HID),
            lp['ln1_g'].reshape(1, HID), lp['ln1_b'].reshape(1, HID),
            lp['e2_W'], lp['e2_b'].reshape(1, HID),
            lp['att_W'], lp['att_b'].reshape(1, 1),
            lp['n1_W'], lp['n1_b'].reshape(1, HID),
            lp['ln2_g'].reshape(1, HID), lp['ln2_b'].reshape(1, HID),
            lp['n2_W'], lp['n2_b'].reshape(1, HID),
        ]
    weights += [p['emb_out_W'], p['emb_out_b'].reshape(1, OUT_F),
                p['out_W'], p['out_b'].reshape(1, 1)]

    data_specs = [
        pl.BlockSpec((V, 3), lambda i: (i, 0)),
        pl.BlockSpec((1, 3 * G, AT), lambda i: (i, 0, 0)),
        pl.BlockSpec((V, IN_F), lambda i: (i, 0)),
        pl.BlockSpec((V, 1), lambda i: (i, 0)),
        pl.BlockSpec((NT, TEMB), lambda i: (0, 0)),
        pl.BlockSpec((1, 1), lambda i: (0, 0)),
        pl.BlockSpec((1, NG), lambda i: (0, 0)),
        pl.BlockSpec((1, NG), lambda i: (0, 0)),
    ]
    w_specs = [pl.BlockSpec(w.shape, lambda i: (0, 0)) for w in weights]

    out = pl.pallas_call(
        _body,
        grid=(NBLK,),
        in_specs=data_specs + w_specs,
        out_specs=pl.BlockSpec((1, G, 1), lambda i: (i, 0, 0)),
        out_shape=jax.ShapeDtypeStruct((NBLK, G, 1), _f32),
        compiler_params=pltpu.CompilerParams(
            dimension_semantics=("parallel",)),
    )(x, ysw, h.astype(_f32), tf, time_table, na, off, coeff,
      *weights)
    return out.reshape(B, 1)


# speedup vs baseline: 1.0379x; 1.0379x over previous
import math

import jax
import jax.numpy as jnp
from jax import lax
from jax.experimental import pallas as pl
from jax.experimental.pallas import tpu as pltpu

N = 50000
AT = 50
B = 1000
K = 8
IN_F = 16
TEMB = 16
HID = 128
OUT_F = 64
NG = 20
DEPTH = 2
NT = 1000

G = 8
V = G * AT
E = V * K
NBLK = B // G

_f32 = jnp.float32
_bf16 = jnp.bfloat16


def _mm(a, b):
    return lax.dot_general(a, b, (((1,), (0,)), ((), ())),
                           preferred_element_type=_f32)


def _hilo(v):
    vh = v.astype(_bf16)
    return vh, v - vh.astype(_f32)


def _mm_sel(oh_b, vals):
    f = vals.shape[1]
    v1, r1 = _hilo(vals)
    v2, r2 = _hilo(r1)
    v3 = r2.astype(_bf16)
    dn = (((1,), (0,)), ((), ()))
    if 3 * f <= 256:
        s = lax.dot_general(oh_b, jnp.concatenate([v1, v2, v3], axis=1),
                            dn, preferred_element_type=_f32)
        return s[:, :f] + s[:, f:2 * f] + s[:, 2 * f:]
    s = lax.dot_general(oh_b, jnp.concatenate([v1, v2], axis=1),
                        dn, preferred_element_type=_f32)
    return (s[:, :f] + s[:, f:]
            + lax.dot_general(oh_b, v3, dn, preferred_element_type=_f32))


def _silu(v):
    return v * jax.nn.sigmoid(v)


def _ln(v, g, b):
    m = jnp.mean(v, axis=-1, keepdims=True)
    d = v - m
    var = jnp.mean(d * d, axis=-1, keepdims=True)
    return d * jax.lax.rsqrt(var + 1e-5) * g + b


def _iota(shape, dim):
    return lax.broadcasted_iota(jnp.int32, shape, dim).astype(_f32)


def _body(*refs):
    xcb = refs[0][...]
    ysw = refs[1][0]
    hb = refs[2][...]
    tfb = refs[3][...]
    tt = refs[4][...]
    na = refs[5][0, 0]
    w = [r[...] for r in refs[8:-1]]
    out_ref = refs[-1]

    rowid = _iota((V, 1), 0)
    ligr = jnp.floor((rowid + 0.5) * (1.0 / AT))
    aloc = rowid - AT * ligr
    iota_ca = _iota((V, AT), 1)
    iota_cv = _iota((V, V), 1)

    iott = _iota((V, NT), 1)
    temb = _mm_sel((iott == tfb).astype(_bf16), tt)

    ohg = (_iota((V, G), 1) == ligr).astype(_bf16)
    d2 = None
    for c in range(3):
        yg = _mm_sel(ohg, ysw[c * G:(c + 1) * G])
        dd = xcb[:, c:c + 1] - yg
        d2 = dd * dd if d2 is None else d2 + dd * dd
    d2m = d2 + jnp.where(iota_ca == aloc, 1e9, 0.0)

    oh_list, rad_list = [], []
    for _ in range(K):
        minv = jnp.min(d2m, axis=1, keepdims=True)
        idx = jnp.min(jnp.where(d2m == minv, iota_ca, 1e9),
                      axis=1, keepdims=True)
        sel = iota_ca == idx
        idxg = idx + AT * ligr
        oh_list.append((iota_cv == idxg).astype(_bf16))
        rad_list.append(minv)
        d2m = jnp.where(sel, 1e30, d2m)
    ohs = jnp.concatenate(oh_list, axis=0)
    radial = jnp.concatenate(rad_list, axis=0)
    keep = [(r < 7.0).astype(_f32) for r in rad_list]

    off = refs[6][...]
    coeff = refs[7][...]
    dc = jnp.clip(radial, 0.0, 4.0) - off
    smear = jnp.exp(coeff * dc * dc)

    it = iter(w)
    emb_in_w, emb_b = next(it), next(it)
    z = _mm(jnp.concatenate([hb, temb], axis=1), emb_in_w) + emb_b

    for _ in range(DEPTH):
        (e1_w, e1_b, ln1_g, ln1_b, e2_w, e2_b, att_w, att_b,
         n1_w, n1_b, ln2_g, ln2_b, n2_w, n2_b) = (
            next(it) for _ in range(14))
        common = (_mm(z, e1_w[0:HID]) + _mm(temb, e1_w[2 * HID + NG:])
                  + e1_b)
        gath = _mm_sel(ohs, _mm(z, e1_w[HID:2 * HID]))
        m = (jnp.concatenate([common] * K, axis=0) + gath
             + _mm(smear, e1_w[2 * HID:2 * HID + NG]))
        m = _silu(_ln(m, ln1_g, ln1_b))
        mij = _silu(_mm(m, e2_w) + e2_b)
        alog = jnp.concatenate(
            [_mm(mij[k * V:(k + 1) * V], att_w) for k in range(K)],
            axis=1) + att_b
        att = jax.nn.sigmoid(alog)
        agg = None
        for k in range(K):
            gk = mij[k * V:(k + 1) * V] * (att[:, k:k + 1] * keep[k])
            agg = gk if agg is None else agg + gk
        agg = agg / 5.0
        o = _silu(_ln(_mm(jnp.concatenate([z, agg], axis=1), n1_w)
                      + n1_b, ln2_g, ln2_b))
        z = z + _mm(o, n2_w) + n2_b

    emb_out_w, emb_out_b, out_w, out_b = (next(it) for _ in range(4))
    zo = _mm(z, emb_out_w) + emb_out_b
    pmat = (_iota((G, V), 0)
            == jnp.floor((_iota((G, V), 1) + 0.5) * (1.0 / AT))
            ).astype(_bf16)
    pooled = _mm_sel(pmat, zo) / na
    out_ref[0] = _mm(pooled, out_w) + out_b


def kernel(x, h, t, num_atoms_per_ligand, batch_ligand, params, time_table):
    xr = x.reshape(NBLK, G, AT, 3)
    ysw = xr.transpose(0, 3, 1, 2).reshape(NBLK, 3 * G, AT)
    tf = t.astype(_f32).reshape(N, 1)
    na = jnp.asarray(num_atoms_per_ligand, _f32).reshape(1, 1)

    off1 = jnp.exp(jnp.linspace(jnp.log(1.0), jnp.log(5.0), NG)) - 1.0
    df = jnp.diff(off1)
    df = jnp.concatenate([df[:1], df])
    coeff1 = -0.5 / df ** 2
    off = off1.reshape(1, NG).astype(_f32)
    coeff = coeff1.reshape(1, NG).astype(_f32)

    p = params
    weights = [p['emb_in_W'], p['emb_in_b'].reshape(1, HID)]
    for lp in p['layers']:
        weights += [
            lp['e1_W'], lp['e1_b'].reshape(1, HID),
            lp['ln1_g'].reshape(1, HID), lp['ln1_b'].reshape(1, HID),
            lp['e2_W'], lp['e2_b'].reshape(1, HID),
            lp['att_W'], lp['att_b'].reshape(1, 1),
            lp['n1_W'], lp['n1_b'].reshape(1, HID),
            lp['ln2_g'].reshape(1, HID), lp['ln2_b'].reshape(1, HID),
            lp['n2_W'], lp['n2_b'].reshape(1, HID),
        ]
    weights += [p['emb_out_W'], p['emb_out_b'].reshape(1, OUT_F),
                p['out_W'], p['out_b'].reshape(1, 1)]

    data_specs = [
        pl.BlockSpec((V, 3), lambda i: (i, 0)),
        pl.BlockSpec((1, 3 * G, AT), lambda i: (i, 0, 0)),
        pl.BlockSpec((V, IN_F), lambda i: (i, 0)),
        pl.BlockSpec((V, 1), lambda i: (i, 0)),
        pl.BlockSpec((NT, TEMB), lambda i: (0, 0)),
        pl.BlockSpec((1, 1), lambda i: (0, 0)),
        pl.BlockSpec((1, NG), lambda i: (0, 0)),
        pl.BlockSpec((1, NG), lambda i: (0, 0)),
    ]
    w_specs = [pl.BlockSpec(w.shape, lambda i: (0, 0)) for w in weights]

    out = pl.pallas_call(
        _body,
        grid=(NBLK,),
        in_specs=data_specs + w_specs,
        out_specs=pl.BlockSpec((1, G, 1), lambda i: (i, 0, 0)),
        out_shape=jax.ShapeDtypeStruct((NBLK, G, 1), _f32),
        compiler_params=pltpu.CompilerParams(
            dimension_semantics=("parallel",)),
    )(x, ysw, h.astype(_f32), tf, time_table, na, off, coeff,
      *weights)
    return out.reshape(B, 1)
